# baseline (speedup 1.0000x reference)
import functools

import jax
import jax.numpy as jnp
from jax import lax
from jax.experimental import pallas as pl
from jax.experimental.pallas import tpu as pltpu
from jax.experimental.pallas import tpu_sc as plsc

B = 4096
F = 26
V = 100000
D = 32
NN = 13

NC = 2
NS = 16
NW = NC * NS
BPW = B // NW
L = 16


def _sc_body(idx_hbm, int_hbm, lin_hbm, s_out, q_out, l_out,
             idx_v, rows_v, lin_v, s_v, q_v, l_v, sem_r, sem_l):
    wid = lax.axis_index("s") * NC + lax.axis_index("c")
    base = wid * BPW
    pltpu.sync_copy(idx_hbm.at[wid], idx_v)
    handles = []
    for f in range(F):
        handles.append(
            pltpu.async_copy(int_hbm.at[idx_v.at[f]], rows_v.at[f], sem_r))
    for f in range(F):
        handles.append(
            pltpu.async_copy(lin_hbm.at[idx_v.at[f]], lin_v.at[f], sem_l))
    for h in handles:
        h.wait()

    for c in range(BPW // L):
        acc = jnp.zeros((L,), jnp.float32)
        for f in range(F):
            acc = acc + lin_v[f, pl.ds(c * L, L)]
        l_v[pl.ds(c * L, L)] = acc

    def body(s, carry):
        s0 = jnp.zeros((L,), jnp.float32)
        s1 = jnp.zeros((L,), jnp.float32)
        q0 = jnp.zeros((L,), jnp.float32)
        q1 = jnp.zeros((L,), jnp.float32)
        for f in range(F):
            r0 = rows_v[f, s, pl.ds(0, L)]
            r1 = rows_v[f, s, pl.ds(L, L)]
            s0 = s0 + r0
            s1 = s1 + r1
            q0 = q0 + r0 * r0
            q1 = q1 + r1 * r1
        s_v[s, pl.ds(0, L)] = s0
        s_v[s, pl.ds(L, L)] = s1
        q_v[s, pl.ds(0, L)] = q0
        q_v[s, pl.ds(L, L)] = q1
        return carry

    lax.fori_loop(0, BPW, body, 0)

    pltpu.sync_copy(s_v, s_out.at[pl.ds(base, BPW)])
    pltpu.sync_copy(q_v, q_out.at[pl.ds(base, BPW)])
    pltpu.sync_copy(l_v, l_out.at[pl.ds(base, BPW)])


_sc_gather = functools.partial(
    pl.kernel,
    out_type=[
        jax.ShapeDtypeStruct((B, D), jnp.float32),
        jax.ShapeDtypeStruct((B, D), jnp.float32),
        jax.ShapeDtypeStruct((B,), jnp.float32),
    ],
    mesh=plsc.VectorSubcoreMesh(core_axis_name="c", subcore_axis_name="s"),
    compiler_params=pltpu.CompilerParams(use_tc_tiling_on_sc=False),
    scratch_types=[
        pltpu.VMEM((F, BPW), jnp.int32),
        pltpu.VMEM((F, BPW, D), jnp.float32),
        pltpu.VMEM((F, BPW), jnp.float32),
        pltpu.VMEM((BPW, D), jnp.float32),
        pltpu.VMEM((BPW, D), jnp.float32),
        pltpu.VMEM((BPW,), jnp.float32),
        pltpu.SemaphoreType.DMA,
        pltpu.SemaphoreType.DMA,
    ],
)(_sc_body)


def _tc_body(s_ref, q_ref, l_ref, xn_ref, nv_ref, b_ref, o_ref):
    xn = xn_ref[...]
    nv = nv_ref[...]
    S = s_ref[...] + jnp.dot(xn, nv, preferred_element_type=jnp.float32,
                             precision=lax.Precision.HIGHEST)
    Q = q_ref[...] + jnp.dot(xn * xn, nv * nv,
                             preferred_element_type=jnp.float32,
                             precision=lax.Precision.HIGHEST)
    inter = 0.5 * (jnp.sum(S * S, axis=1, keepdims=True)
                   - jnp.sum(Q, axis=1, keepdims=True))
    o_ref[...] = l_ref[...] + b_ref[0] + inter


_tc_combine = pl.pallas_call(
    _tc_body,
    out_shape=jax.ShapeDtypeStruct((B, 1), jnp.float32),
    in_specs=[
        pl.BlockSpec(memory_space=pltpu.VMEM),
        pl.BlockSpec(memory_space=pltpu.VMEM),
        pl.BlockSpec(memory_space=pltpu.VMEM),
        pl.BlockSpec(memory_space=pltpu.VMEM),
        pl.BlockSpec(memory_space=pltpu.VMEM),
        pl.BlockSpec(memory_space=pltpu.SMEM),
    ],
)


def kernel(x_numeric, x_categorical, lin_tables, int_tables, num_vecs, bias):
    int_flat = int_tables.reshape(F * V, D)
    lin_flat = lin_tables.reshape(F * V)
    offs = (jnp.arange(F, dtype=jnp.int32) * V)[None, :]
    flat_idx = x_categorical + offs
    idx3 = flat_idx.reshape(NW, BPW, F).transpose(0, 2, 1)
    s_cat, q_cat, lin_sum = _sc_gather(idx3, int_flat, lin_flat)
    out = _tc_combine(s_cat, q_cat, lin_sum[:, None], x_numeric, num_vecs,
                      bias)
    return out[:, 0]


# speedup vs baseline: 1.0004x; 1.0004x over previous
import functools

import jax
import jax.numpy as jnp
from jax import lax
from jax.experimental import pallas as pl
from jax.experimental.pallas import tpu as pltpu
from jax.experimental.pallas import tpu_sc as plsc

B = 4096
F = 26
V = 100000
D = 32
NN = 13

NC = 2
NS = 16
NW = NC * NS
BPW = B // NW
L = 16


def _sc_body(idx_hbm, int_hbm, lin_hbm, s_out, q_out, l_out,
             idx_v, rows_v, lin_v, s_v, q_v, l_v, sem_r, sem_l):
    wid = lax.axis_index("s") * NC + lax.axis_index("c")
    base = wid * BPW
    pltpu.sync_copy(idx_hbm.at[wid], idx_v)
    handles = []
    for f in range(F):
        handles.append(
            pltpu.async_copy(int_hbm.at[idx_v.at[f]], rows_v.at[f], sem_r))
    for f in range(F):
        handles.append(
            pltpu.async_copy(lin_hbm.at[idx_v.at[f]], lin_v.at[f], sem_l))
    for h in handles:
        h.wait()

    for c in range(BPW // L):
        acc = jnp.zeros((L,), jnp.float32)
        for f in range(F):
            acc = acc + lin_v[f, pl.ds(c * L, L)]
        l_v[pl.ds(c * L, L)] = acc

    def body(s, carry):
        s0 = jnp.zeros((L,), jnp.float32)
        s1 = jnp.zeros((L,), jnp.float32)
        q0 = jnp.zeros((L,), jnp.float32)
        q1 = jnp.zeros((L,), jnp.float32)
        for f in range(F):
            r0 = rows_v[f, s, pl.ds(0, L)]
            r1 = rows_v[f, s, pl.ds(L, L)]
            s0 = s0 + r0
            s1 = s1 + r1
            q0 = q0 + r0 * r0
            q1 = q1 + r1 * r1
        s_v[s, pl.ds(0, L)] = s0
        s_v[s, pl.ds(L, L)] = s1
        q_v[s, pl.ds(0, L)] = q0
        q_v[s, pl.ds(L, L)] = q1
        return carry

    lax.fori_loop(0, BPW, body, 0)

    pltpu.sync_copy(s_v, s_out.at[pl.ds(base, BPW)])
    pltpu.sync_copy(q_v, q_out.at[pl.ds(base, BPW)])
    pltpu.sync_copy(l_v, l_out.at[pl.ds(base, BPW)])


_sc_gather = functools.partial(
    pl.kernel,
    out_type=[
        jax.ShapeDtypeStruct((B, D), jnp.float32),
        jax.ShapeDtypeStruct((B, D), jnp.float32),
        jax.ShapeDtypeStruct((B,), jnp.float32),
    ],
    mesh=plsc.VectorSubcoreMesh(core_axis_name="c", subcore_axis_name="s"),
    compiler_params=pltpu.CompilerParams(use_tc_tiling_on_sc=False),
    scratch_types=[
        pltpu.VMEM((F, BPW), jnp.int32),
        pltpu.VMEM((F, BPW, D), jnp.float32),
        pltpu.VMEM((F, BPW), jnp.float32),
        pltpu.VMEM((BPW, D), jnp.float32),
        pltpu.VMEM((BPW, D), jnp.float32),
        pltpu.VMEM((BPW,), jnp.float32),
        pltpu.SemaphoreType.DMA,
        pltpu.SemaphoreType.DMA,
    ],
)(_sc_body)


def _tc_body(s_ref, q_ref, l_ref, xn_ref, nv_ref, b_ref, o_ref):
    xn = xn_ref[...]
    nv = nv_ref[...]
    S = s_ref[...] + jnp.dot(xn, nv, preferred_element_type=jnp.float32,
                             precision=lax.Precision.HIGHEST)
    Q = q_ref[...] + jnp.dot(xn * xn, nv * nv,
                             preferred_element_type=jnp.float32,
                             precision=lax.Precision.HIGHEST)
    inter = 0.5 * jnp.sum(S * S - Q, axis=1, keepdims=True)
    o_ref[...] = l_ref[...] + b_ref[0] + inter


_tc_combine = pl.pallas_call(
    _tc_body,
    out_shape=jax.ShapeDtypeStruct((B, 1), jnp.float32),
    in_specs=[
        pl.BlockSpec(memory_space=pltpu.VMEM),
        pl.BlockSpec(memory_space=pltpu.VMEM),
        pl.BlockSpec(memory_space=pltpu.VMEM),
        pl.BlockSpec(memory_space=pltpu.VMEM),
        pl.BlockSpec(memory_space=pltpu.VMEM),
        pl.BlockSpec(memory_space=pltpu.SMEM),
    ],
)


def kernel(x_numeric, x_categorical, lin_tables, int_tables, num_vecs, bias):
    int_flat = int_tables.reshape(F * V, D)
    lin_flat = lin_tables.reshape(F * V)
    offs = (jnp.arange(F, dtype=jnp.int32) * V)[None, :]
    flat_idx = x_categorical + offs
    idx3 = flat_idx.reshape(NW, BPW, F).transpose(0, 2, 1)
    s_cat, q_cat, lin_sum = _sc_gather(idx3, int_flat, lin_flat)
    out = _tc_combine(s_cat, q_cat, lin_sum[:, None], x_numeric, num_vecs,
                      bias)
    return out[:, 0]
